# baseline (speedup 1.0000x reference)
import jax
import jax.numpy as jnp
from jax.experimental import pallas as pl
from jax.experimental.pallas import tpu as pltpu

B = 4096
D = 1024
H = 2048
E = 8
K = 4
BH = 1024
NJ = H // BH


def _moe_kernel(x_ref, wg_ref, b2_ref, w1_ref, b1_ref, w2_ref,
                out_ref, xb_ref, gates_ref):
    e = pl.program_id(0)
    j = pl.program_id(1)

    @pl.when((e == 0) & (j == 0))
    def _route():
        x = x_ref[...]
        l = jax.lax.dot_general(
            x, wg_ref[...], (((1,), (0,)), ((), ())),
            preferred_element_type=jnp.float32)
        lt = l.T
        ei = jax.lax.broadcasted_iota(jnp.int32, (E, B), 0)
        rank = jnp.zeros((E, B), jnp.int32)
        for jj in range(E):
            lj = lt[jj:jj + 1, :]
            beats = (lj > lt) | ((lj == lt) & (jj < ei))
            rank = rank + beats.astype(jnp.int32)
        sel = rank < K
        m = jnp.max(lt, axis=0, keepdims=True)
        ex = jnp.where(sel, jnp.exp(lt - m), 0.0)
        g = (ex / jnp.sum(ex, axis=0, keepdims=True)).T
        gates_ref[...] = g
        out_ref[...] = jnp.dot(g, b2_ref[...],
                               preferred_element_type=jnp.float32)
        xb_ref[...] = x.astype(jnp.bfloat16)

    w1b = w1_ref[0].astype(jnp.bfloat16)
    h = jnp.dot(xb_ref[...], w1b, preferred_element_type=jnp.float32)
    h = jnp.maximum(h + b1_ref[0], 0.0)
    partial = jnp.sum(h * w2_ref[0], axis=1, keepdims=True)
    onehot = (jax.lax.broadcasted_iota(jnp.int32, (E, 1), 0) == e
              ).astype(jnp.float32)
    g = jnp.dot(gates_ref[...], onehot, preferred_element_type=jnp.float32)
    out_ref[...] += g * partial


def kernel(x, w_gate, W1, b1, W2, b2):
    b1r = b1.reshape(E, 1, H)
    W2r = W2.reshape(E, 1, H)

    out = pl.pallas_call(
        _moe_kernel,
        grid=(E, NJ),
        in_specs=[
            pl.BlockSpec((B, D), lambda e, j: (0, 0)),
            pl.BlockSpec((D, E), lambda e, j: (0, 0)),
            pl.BlockSpec((E, 1), lambda e, j: (0, 0)),
            pl.BlockSpec((1, D, BH), lambda e, j: (e, 0, j)),
            pl.BlockSpec((1, 1, BH), lambda e, j: (e, 0, j)),
            pl.BlockSpec((1, 1, BH), lambda e, j: (e, 0, j)),
        ],
        out_specs=pl.BlockSpec((B, 1), lambda e, j: (0, 0)),
        out_shape=jax.ShapeDtypeStruct((B, 1), jnp.float32),
        scratch_shapes=[
            pltpu.VMEM((B, D), jnp.bfloat16),
            pltpu.VMEM((B, E), jnp.float32),
        ],
        compiler_params=pltpu.CompilerParams(
            dimension_semantics=("arbitrary", "arbitrary")),
    )(x, w_gate, b2, W1, b1r, W2r)
    return out
